# baseline (speedup 1.0000x reference)
import functools

import jax
import jax.numpy as jnp
from jax import lax
from jax.experimental import pallas as pl
from jax.experimental.pallas import tpu as pltpu
from jax.experimental.pallas import tpu_sc as plsc

B = 1024
D = 100000
H = 64


_NC = 2
_NS = 16
_NW = _NC * _NS
_BPW = B // _NW


def _sc_gather_body(table_hbm, idx_hbm, out_hbm, idx_v, rows_v, sem):
    wid = lax.axis_index("s") * _NC + lax.axis_index("c")
    base = wid * _BPW
    pltpu.sync_copy(idx_hbm.at[pl.ds(base, _BPW)], idx_v)
    pltpu.async_copy(table_hbm.at[idx_v], rows_v, sem).wait()
    pltpu.sync_copy(rows_v, out_hbm.at[pl.ds(base, _BPW)])


_sc_gather = pl.kernel(
    _sc_gather_body,
    out_type=jax.ShapeDtypeStruct((B, H), jnp.float32),
    mesh=plsc.VectorSubcoreMesh(core_axis_name="c", subcore_axis_name="s"),
    scratch_types=[
        pltpu.VMEM((_BPW,), jnp.int32),
        pltpu.VMEM((_BPW, H), jnp.float32),
        pltpu.SemaphoreType.DMA,
    ],
    compiler_params=pltpu.CompilerParams(use_tc_tiling_on_sc=False),
)


_BLK = 2048


def _proj_body(emb_ref, bias_ref, w_ref, b_ref, out_ref):
    e = emb_ref[...] + bias_ref[...]
    out_ref[...] = (
        lax.dot_general(
            e, w_ref[...], (((1,), (1,)), ((), ())),
            preferred_element_type=jnp.float32,
        )
        + b_ref[...]
    )


@jax.jit
def kernel(x, weight, bias, W_out, b_out):
    emb = _sc_gather(weight, x.astype(jnp.int32))

    grid = (pl.cdiv(D, _BLK),)
    out = pl.pallas_call(
        _proj_body,
        grid=grid,
        in_specs=[
            pl.BlockSpec((B, H), lambda j: (0, 0)),
            pl.BlockSpec((1, H), lambda j: (0, 0)),
            pl.BlockSpec((_BLK, H), lambda j: (j, 0)),
            pl.BlockSpec((1, _BLK), lambda j: (0, j)),
        ],
        out_specs=pl.BlockSpec((B, _BLK), lambda j: (0, j)),
        out_shape=jax.ShapeDtypeStruct((B, D), jnp.float32),
    )(emb, bias.reshape(1, H), W_out, b_out.reshape(1, D))
    return out


# speedup vs baseline: 1.0008x; 1.0008x over previous
import jax
import jax.numpy as jnp
from jax import lax
from jax.experimental import pallas as pl
from jax.experimental.pallas import tpu as pltpu
from jax.experimental.pallas import tpu_sc as plsc

B = 1024
D = 100000
H = 64


_NC = 2
_NS = 16
_NW = _NC * _NS
_BPW = B // _NW


def _sc_gather_body(table_hbm, idx_hbm, out_hbm, idx_v, rows_v, sem):
    wid = lax.axis_index("s") * _NC + lax.axis_index("c")
    base = wid * _BPW
    pltpu.sync_copy(idx_hbm.at[pl.ds(base, _BPW)], idx_v)
    pltpu.async_copy(table_hbm.at[idx_v], rows_v, sem).wait()
    pltpu.sync_copy(rows_v, out_hbm.at[pl.ds(base, _BPW)])


_sc_gather = pl.kernel(
    _sc_gather_body,
    out_type=jax.ShapeDtypeStruct((B, 2 * H), jnp.float32),
    mesh=plsc.VectorSubcoreMesh(core_axis_name="c", subcore_axis_name="s"),
    scratch_types=[
        pltpu.VMEM((_BPW,), jnp.int32),
        pltpu.VMEM((_BPW, 2 * H), jnp.float32),
        pltpu.SemaphoreType.DMA,
    ],
)


_BLK = 2048


def _proj_body(emb_ref, par_ref, bias_ref, w_ref, b_ref, out_ref):
    lo = emb_ref[:, :H]
    hi = emb_ref[:, H:]
    e = jnp.where(par_ref[...] > 0, hi, lo) + bias_ref[...]
    out_ref[...] = (
        lax.dot_general(
            e, w_ref[...], (((1,), (1,)), ((), ())),
            preferred_element_type=jnp.float32,
        )
        + b_ref[...]
    )


@jax.jit
def kernel(x, weight, bias, W_out, b_out):
    x = x.astype(jnp.int32)
    table2 = weight.reshape(D // 2, 2 * H)
    emb128 = _sc_gather(table2, x >> 1)
    par = (x & 1).reshape(B, 1)

    grid = (pl.cdiv(D, _BLK),)
    out = pl.pallas_call(
        _proj_body,
        grid=grid,
        in_specs=[
            pl.BlockSpec((B, 2 * H), lambda j: (0, 0)),
            pl.BlockSpec((B, 1), lambda j: (0, 0)),
            pl.BlockSpec((1, H), lambda j: (0, 0)),
            pl.BlockSpec((_BLK, H), lambda j: (j, 0)),
            pl.BlockSpec((1, _BLK), lambda j: (0, j)),
        ],
        out_specs=pl.BlockSpec((B, _BLK), lambda j: (0, j)),
        out_shape=jax.ShapeDtypeStruct((B, D), jnp.float32),
    )(emb128, par, bias.reshape(1, H), W_out, b_out.reshape(1, D))
    return out


# speedup vs baseline: 2.8371x; 2.8348x over previous
import jax
import jax.numpy as jnp
from jax import lax
from jax.experimental import pallas as pl
from jax.experimental.pallas import tpu as pltpu
from jax.experimental.pallas import tpu_sc as plsc

B = 1024
D = 100000
H = 64


_NC = 2
_NS = 16
_NW = _NC * _NS
_BPW = B // _NW


def _sc_gather_body(table_hbm, idx_hbm, out_hbm, idx_v, rows_v, sem):
    wid = lax.axis_index("s") * _NC + lax.axis_index("c")
    base = wid * _BPW
    pltpu.sync_copy(idx_hbm.at[pl.ds(base, _BPW)], idx_v)
    pltpu.async_copy(table_hbm.at[idx_v], rows_v, sem).wait()
    pltpu.sync_copy(rows_v, out_hbm.at[pl.ds(base, _BPW)])


_sc_gather = pl.kernel(
    _sc_gather_body,
    out_type=jax.ShapeDtypeStruct((B, 2 * H), jnp.float32),
    mesh=plsc.VectorSubcoreMesh(core_axis_name="c", subcore_axis_name="s"),
    scratch_types=[
        pltpu.VMEM((_BPW,), jnp.int32),
        pltpu.VMEM((_BPW, 2 * H), jnp.float32),
        pltpu.SemaphoreType.DMA,
    ],
)


_BLK = 2048


def _proj_body(emb_ref, par_ref, bias_ref, wt_ref, b_ref, out_ref):
    lo = emb_ref[:, :H]
    hi = emb_ref[:, H:]
    e = jnp.where(par_ref[...] > 0, hi, lo) + bias_ref[...]
    e1 = jnp.concatenate([e, jnp.ones((B, 1), jnp.float32)], axis=1)
    w1 = jnp.concatenate([wt_ref[...], b_ref[...]], axis=0)
    out_ref[...] = lax.dot_general(
        w1, e1, (((0,), (1,)), ((), ())),
        preferred_element_type=jnp.float32,
    )


@jax.jit
def kernel(x, weight, bias, W_out, b_out):
    x = x.astype(jnp.int32)
    table2 = weight.reshape(D // 2, 2 * H)
    emb128 = _sc_gather(table2, x >> 1)
    par = (x & 1).reshape(B, 1)
    w_t = jnp.swapaxes(W_out, 0, 1)

    grid = (pl.cdiv(D, _BLK),)
    out_t = pl.pallas_call(
        _proj_body,
        grid=grid,
        in_specs=[
            pl.BlockSpec((B, 2 * H), lambda j: (0, 0)),
            pl.BlockSpec((B, 1), lambda j: (0, 0)),
            pl.BlockSpec((1, H), lambda j: (0, 0)),
            pl.BlockSpec((H, _BLK), lambda j: (0, j)),
            pl.BlockSpec((1, _BLK), lambda j: (0, j)),
        ],
        out_specs=pl.BlockSpec((_BLK, B), lambda j: (j, 0)),
        out_shape=jax.ShapeDtypeStruct((D, B), jnp.float32),
    )(emb128, par, bias.reshape(1, H), w_t, b_out.reshape(1, D))
    return jnp.swapaxes(out_t, 0, 1)


# speedup vs baseline: 3.0743x; 1.0836x over previous
import jax
import jax.numpy as jnp
from jax import lax
from jax.experimental import pallas as pl
from jax.experimental.pallas import tpu as pltpu
from jax.experimental.pallas import tpu_sc as plsc

B = 1024
D = 100000
H = 64


_RBLK = 2048


def _retile_body(wt_ref, out_ref):
    t = jnp.transpose(wt_ref[...], (1, 0))
    out_ref[:, :H] = t
    out_ref[:, H:] = jnp.zeros((_RBLK, H), jnp.float32)


_NC = 2
_NS = 16
_NW = _NC * _NS
_BPW = B // _NW


def _sc_gather_body(table_hbm, idx_hbm, out_hbm, idx_v, rows_v, sem):
    wid = lax.axis_index("s") * _NC + lax.axis_index("c")
    base = wid * _BPW
    pltpu.sync_copy(idx_hbm.at[pl.ds(base, _BPW)], idx_v)
    pltpu.async_copy(table_hbm.at[idx_v], rows_v, sem).wait()
    pltpu.sync_copy(rows_v, out_hbm.at[pl.ds(base, _BPW)])


_sc_gather = pl.kernel(
    _sc_gather_body,
    out_type=jax.ShapeDtypeStruct((B, 2 * H), jnp.float32),
    mesh=plsc.VectorSubcoreMesh(core_axis_name="c", subcore_axis_name="s"),
    scratch_types=[
        pltpu.VMEM((_BPW,), jnp.int32),
        pltpu.VMEM((_BPW, 2 * H), jnp.float32),
        pltpu.SemaphoreType.DMA,
    ],
)


_BLK = 2048


def _proj_body(emb_ref, bias_ref, wt_ref, b_ref, out_ref):
    e = emb_ref[:, :H] + bias_ref[...]
    e1 = jnp.concatenate([e, jnp.ones((B, 1), jnp.float32)], axis=1)
    w1 = jnp.concatenate([wt_ref[...], b_ref[...]], axis=0)
    out_ref[...] = lax.dot_general(
        w1, e1, (((0,), (1,)), ((), ())),
        preferred_element_type=jnp.float32,
    )


@jax.jit
def kernel(x, weight, bias, W_out, b_out):
    x = x.astype(jnp.int32)
    weight_t = jnp.swapaxes(weight, 0, 1)
    w_t = jnp.swapaxes(W_out, 0, 1)

    table_pad = pl.pallas_call(
        _retile_body,
        grid=(pl.cdiv(D, _RBLK),),
        in_specs=[pl.BlockSpec((H, _RBLK), lambda j: (0, j))],
        out_specs=pl.BlockSpec((_RBLK, 2 * H), lambda j: (j, 0)),
        out_shape=jax.ShapeDtypeStruct((D, 2 * H), jnp.float32),
    )(weight_t)

    emb128 = _sc_gather(table_pad, x)

    out_t = pl.pallas_call(
        _proj_body,
        grid=(pl.cdiv(D, _BLK),),
        in_specs=[
            pl.BlockSpec((B, 2 * H), lambda j: (0, 0)),
            pl.BlockSpec((1, H), lambda j: (0, 0)),
            pl.BlockSpec((H, _BLK), lambda j: (0, j)),
            pl.BlockSpec((1, _BLK), lambda j: (0, j)),
        ],
        out_specs=pl.BlockSpec((_BLK, B), lambda j: (j, 0)),
        out_shape=jax.ShapeDtypeStruct((D, B), jnp.float32),
    )(emb128, bias.reshape(1, H), w_t, b_out.reshape(1, D))
    return jnp.swapaxes(out_t, 0, 1)


# speedup vs baseline: 3.2924x; 1.0710x over previous
import jax
import jax.numpy as jnp
from jax import lax
from jax.experimental import pallas as pl
from jax.experimental.pallas import tpu as pltpu
from jax.experimental.pallas import tpu_sc as plsc

B = 1024
D = 100000
H = 64


_RBLK = 4096


def _retile_body(wt_ref, out_ref):
    t = jnp.transpose(wt_ref[...], (1, 0))
    out_ref[:, :H] = t
    out_ref[:, H:] = jnp.zeros((_RBLK, H), jnp.float32)


_NC = 2
_NS = 16
_NW = _NC * _NS
_BPW = B // _NW


def _sc_gather_body(table_hbm, idx_hbm, out_hbm, idx_v, rows_v, sem):
    wid = lax.axis_index("s") * _NC + lax.axis_index("c")
    base = wid * _BPW
    pltpu.sync_copy(idx_hbm.at[pl.ds(base, _BPW)], idx_v)
    pltpu.async_copy(table_hbm.at[idx_v], rows_v, sem).wait()
    pltpu.sync_copy(rows_v, out_hbm.at[pl.ds(base, _BPW)])


_sc_gather = pl.kernel(
    _sc_gather_body,
    out_type=jax.ShapeDtypeStruct((B, 2 * H), jnp.float32),
    mesh=plsc.VectorSubcoreMesh(core_axis_name="c", subcore_axis_name="s"),
    scratch_types=[
        pltpu.VMEM((_BPW,), jnp.int32),
        pltpu.VMEM((_BPW, 2 * H), jnp.float32),
        pltpu.SemaphoreType.DMA,
    ],
)


_BLK = 2048


def _proj_body(emb_ref, bias_ref, wt_ref, b_ref, out_ref):
    e = emb_ref[:, :H] + bias_ref[...]
    e1 = jnp.concatenate([e, jnp.ones((B, 1), jnp.float32)], axis=1)
    w1 = jnp.concatenate([wt_ref[...], b_ref[...]], axis=0)
    out_ref[...] = lax.dot_general(
        w1, e1, (((0,), (1,)), ((), ())),
        preferred_element_type=jnp.float32,
    )


@jax.jit
def kernel(x, weight, bias, W_out, b_out):
    x = x.astype(jnp.int32)
    weight_t = jnp.swapaxes(weight, 0, 1)
    w_t = jnp.swapaxes(W_out, 0, 1)

    table_pad = pl.pallas_call(
        _retile_body,
        grid=(pl.cdiv(D, _RBLK),),
        in_specs=[pl.BlockSpec((H, _RBLK), lambda j: (0, j))],
        out_specs=pl.BlockSpec((_RBLK, 2 * H), lambda j: (j, 0)),
        out_shape=jax.ShapeDtypeStruct((D, 2 * H), jnp.float32),
    )(weight_t)

    emb128 = _sc_gather(table_pad, x)

    out_t = pl.pallas_call(
        _proj_body,
        grid=(pl.cdiv(D, _BLK),),
        in_specs=[
            pl.BlockSpec((B, 2 * H), lambda j: (0, 0)),
            pl.BlockSpec((1, H), lambda j: (0, 0)),
            pl.BlockSpec((H, _BLK), lambda j: (0, j)),
            pl.BlockSpec((1, _BLK), lambda j: (0, j)),
        ],
        out_specs=pl.BlockSpec((_BLK, B), lambda j: (j, 0)),
        out_shape=jax.ShapeDtypeStruct((D, B), jnp.float32),
    )(emb128, bias.reshape(1, H), w_t, b_out.reshape(1, D))
    return jnp.swapaxes(out_t, 0, 1)


# speedup vs baseline: 3.4002x; 1.0327x over previous
import jax
import jax.numpy as jnp
from jax import lax
from jax.experimental import pallas as pl
from jax.experimental.pallas import tpu as pltpu
from jax.experimental.pallas import tpu_sc as plsc

B = 1024
D = 100000
H = 64


_RBLK = 8192


def _retile_body(wt_ref, out_ref):
    t = jnp.transpose(wt_ref[...], (1, 0))
    out_ref[:, :H] = t
    out_ref[:, H:] = jnp.zeros((_RBLK, H), jnp.float32)


_NC = 2
_NS = 16
_NW = _NC * _NS
_BPW = B // _NW


def _sc_gather_body(table_hbm, idx_hbm, out_hbm, idx_v, rows_v, sem):
    wid = lax.axis_index("s") * _NC + lax.axis_index("c")
    base = wid * _BPW
    pltpu.sync_copy(idx_hbm.at[pl.ds(base, _BPW)], idx_v)
    pltpu.async_copy(table_hbm.at[idx_v], rows_v, sem).wait()
    pltpu.sync_copy(rows_v, out_hbm.at[pl.ds(base, _BPW)])


_sc_gather = pl.kernel(
    _sc_gather_body,
    out_type=jax.ShapeDtypeStruct((B, 2 * H), jnp.float32),
    mesh=plsc.VectorSubcoreMesh(core_axis_name="c", subcore_axis_name="s"),
    scratch_types=[
        pltpu.VMEM((_BPW,), jnp.int32),
        pltpu.VMEM((_BPW, 2 * H), jnp.float32),
        pltpu.SemaphoreType.DMA,
    ],
)


_BLK = 2048


def _proj_body(emb_ref, bias_ref, wt_ref, b_ref, out_ref):
    e = emb_ref[:, :H] + bias_ref[...]
    e1 = jnp.concatenate([e, jnp.ones((B, 1), jnp.float32)], axis=1)
    w1 = jnp.concatenate([wt_ref[...], b_ref[...]], axis=0)
    out_ref[...] = lax.dot_general(
        w1, e1, (((0,), (1,)), ((), ())),
        preferred_element_type=jnp.float32,
    )


@jax.jit
def kernel(x, weight, bias, W_out, b_out):
    x = x.astype(jnp.int32)
    weight_t = jnp.swapaxes(weight, 0, 1)
    w_t = jnp.swapaxes(W_out, 0, 1)

    table_pad = pl.pallas_call(
        _retile_body,
        grid=(pl.cdiv(D, _RBLK),),
        in_specs=[pl.BlockSpec((H, _RBLK), lambda j: (0, j))],
        out_specs=pl.BlockSpec((_RBLK, 2 * H), lambda j: (j, 0)),
        out_shape=jax.ShapeDtypeStruct((D, 2 * H), jnp.float32),
    )(weight_t)

    emb128 = _sc_gather(table_pad, x)

    out_t = pl.pallas_call(
        _proj_body,
        grid=(pl.cdiv(D, _BLK),),
        in_specs=[
            pl.BlockSpec((B, 2 * H), lambda j: (0, 0)),
            pl.BlockSpec((1, H), lambda j: (0, 0)),
            pl.BlockSpec((H, _BLK), lambda j: (0, j)),
            pl.BlockSpec((1, _BLK), lambda j: (0, j)),
        ],
        out_specs=pl.BlockSpec((_BLK, B), lambda j: (j, 0)),
        out_shape=jax.ShapeDtypeStruct((D, B), jnp.float32),
    )(emb128, bias.reshape(1, H), w_t, b_out.reshape(1, D))
    return jnp.swapaxes(out_t, 0, 1)


# speedup vs baseline: 3.4402x; 1.0118x over previous
import jax
import jax.numpy as jnp
from jax import lax
from jax.experimental import pallas as pl
from jax.experimental.pallas import tpu as pltpu
from jax.experimental.pallas import tpu_sc as plsc

B = 1024
D = 100000
H = 64


_RBLK = 8192


def _retile_body(wt_ref, out_ref):
    t = jnp.transpose(wt_ref[...], (1, 0))
    out_ref[:, :H] = t
    out_ref[:, H:] = jnp.zeros((_RBLK, H), jnp.float32)


_NC = 2
_NS = 16
_NW = _NC * _NS
_BPW = B // _NW


def _sc_gather_body(table_hbm, idx_hbm, out_hbm, idx_v, rows_v, sem):
    wid = lax.axis_index("s") * _NC + lax.axis_index("c")
    base = wid * _BPW
    pltpu.sync_copy(idx_hbm.at[pl.ds(base, _BPW)], idx_v)
    pltpu.async_copy(table_hbm.at[idx_v], rows_v, sem).wait()
    pltpu.sync_copy(rows_v, out_hbm.at[pl.ds(base, _BPW)])


_sc_gather = pl.kernel(
    _sc_gather_body,
    out_type=jax.ShapeDtypeStruct((B, 2 * H), jnp.float32),
    mesh=plsc.VectorSubcoreMesh(core_axis_name="c", subcore_axis_name="s"),
    scratch_types=[
        pltpu.VMEM((_BPW,), jnp.int32),
        pltpu.VMEM((_BPW, 2 * H), jnp.float32),
        pltpu.SemaphoreType.DMA,
    ],
)


_BLK = 4096


def _proj_body(emb_ref, bias_ref, wt_ref, b_ref, out_ref):
    e = emb_ref[:, :H] + bias_ref[...]
    e1 = jnp.concatenate([e, jnp.ones((B, 1), jnp.float32)], axis=1)
    w1 = jnp.concatenate([wt_ref[...], b_ref[...]], axis=0)
    out_ref[...] = lax.dot_general(
        w1, e1, (((0,), (1,)), ((), ())),
        preferred_element_type=jnp.float32,
    )


@jax.jit
def kernel(x, weight, bias, W_out, b_out):
    x = x.astype(jnp.int32)
    weight_t = jnp.swapaxes(weight, 0, 1)
    w_t = jnp.swapaxes(W_out, 0, 1)

    table_pad = pl.pallas_call(
        _retile_body,
        grid=(pl.cdiv(D, _RBLK),),
        in_specs=[pl.BlockSpec((H, _RBLK), lambda j: (0, j))],
        out_specs=pl.BlockSpec((_RBLK, 2 * H), lambda j: (j, 0)),
        out_shape=jax.ShapeDtypeStruct((D, 2 * H), jnp.float32),
    )(weight_t)

    emb128 = _sc_gather(table_pad, x)

    out_t = pl.pallas_call(
        _proj_body,
        grid=(pl.cdiv(D, _BLK),),
        in_specs=[
            pl.BlockSpec((B, 2 * H), lambda j: (0, 0)),
            pl.BlockSpec((1, H), lambda j: (0, 0)),
            pl.BlockSpec((H, _BLK), lambda j: (0, j)),
            pl.BlockSpec((1, _BLK), lambda j: (0, j)),
        ],
        out_specs=pl.BlockSpec((_BLK, B), lambda j: (j, 0)),
        out_shape=jax.ShapeDtypeStruct((D, B), jnp.float32),
    )(emb128, bias.reshape(1, H), w_t, b_out.reshape(1, D))
    return jnp.swapaxes(out_t, 0, 1)


# speedup vs baseline: 3.4661x; 1.0075x over previous
import jax
import jax.numpy as jnp
from jax import lax
from jax.experimental import pallas as pl
from jax.experimental.pallas import tpu as pltpu
from jax.experimental.pallas import tpu_sc as plsc

B = 1024
D = 100000
H = 64


_RBLK = 16384


def _retile_body(wt_ref, out_ref):
    t = jnp.transpose(wt_ref[...], (1, 0))
    out_ref[:, :H] = t
    out_ref[:, H:] = jnp.zeros((_RBLK, H), jnp.float32)


_NC = 2
_NS = 16
_NW = _NC * _NS
_BPW = B // _NW


def _sc_gather_body(table_hbm, idx_hbm, out_hbm, idx_v, rows_v, sem):
    wid = lax.axis_index("s") * _NC + lax.axis_index("c")
    base = wid * _BPW
    pltpu.sync_copy(idx_hbm.at[pl.ds(base, _BPW)], idx_v)
    pltpu.async_copy(table_hbm.at[idx_v], rows_v, sem).wait()
    pltpu.sync_copy(rows_v, out_hbm.at[pl.ds(base, _BPW)])


_sc_gather = pl.kernel(
    _sc_gather_body,
    out_type=jax.ShapeDtypeStruct((B, 2 * H), jnp.float32),
    mesh=plsc.VectorSubcoreMesh(core_axis_name="c", subcore_axis_name="s"),
    scratch_types=[
        pltpu.VMEM((_BPW,), jnp.int32),
        pltpu.VMEM((_BPW, 2 * H), jnp.float32),
        pltpu.SemaphoreType.DMA,
    ],
)


_BLK = 4096


def _proj_body(emb_ref, bias_ref, wt_ref, b_ref, out_ref):
    e = emb_ref[:, :H] + bias_ref[...]
    e1 = jnp.concatenate([e, jnp.ones((B, 1), jnp.float32)], axis=1)
    w1 = jnp.concatenate([wt_ref[...], b_ref[...]], axis=0)
    out_ref[...] = lax.dot_general(
        w1, e1, (((0,), (1,)), ((), ())),
        preferred_element_type=jnp.float32,
    )


@jax.jit
def kernel(x, weight, bias, W_out, b_out):
    x = x.astype(jnp.int32)
    weight_t = jnp.swapaxes(weight, 0, 1)
    w_t = jnp.swapaxes(W_out, 0, 1)

    table_pad = pl.pallas_call(
        _retile_body,
        grid=(pl.cdiv(D, _RBLK),),
        in_specs=[pl.BlockSpec((H, _RBLK), lambda j: (0, j))],
        out_specs=pl.BlockSpec((_RBLK, 2 * H), lambda j: (j, 0)),
        out_shape=jax.ShapeDtypeStruct((D, 2 * H), jnp.float32),
    )(weight_t)

    emb128 = _sc_gather(table_pad, x)

    out_t = pl.pallas_call(
        _proj_body,
        grid=(pl.cdiv(D, _BLK),),
        in_specs=[
            pl.BlockSpec((B, 2 * H), lambda j: (0, 0)),
            pl.BlockSpec((1, H), lambda j: (0, 0)),
            pl.BlockSpec((H, _BLK), lambda j: (0, j)),
            pl.BlockSpec((1, _BLK), lambda j: (0, j)),
        ],
        out_specs=pl.BlockSpec((_BLK, B), lambda j: (j, 0)),
        out_shape=jax.ShapeDtypeStruct((D, B), jnp.float32),
    )(emb128, bias.reshape(1, H), w_t, b_out.reshape(1, D))
    return jnp.swapaxes(out_t, 0, 1)


# speedup vs baseline: 3.4715x; 1.0016x over previous
import jax
import jax.numpy as jnp
from jax import lax
from jax.experimental import pallas as pl
from jax.experimental.pallas import tpu as pltpu
from jax.experimental.pallas import tpu_sc as plsc

B = 1024
D = 100000
H = 64


_RBLK = 16384


def _retile_body(wt_ref, out_ref):
    t = jnp.transpose(wt_ref[...], (1, 0))
    out_ref[:, :H] = t
    out_ref[:, H:] = jnp.zeros((_RBLK, H), jnp.float32)


_NC = 2
_NS = 16
_NW = _NC * _NS
_BPW = B // _NW


def _sc_gather_body(table_hbm, idx_hbm, out_hbm, idx_v, rows_v, sem):
    wid = lax.axis_index("s") * _NC + lax.axis_index("c")
    base = wid * _BPW
    pltpu.sync_copy(idx_hbm.at[pl.ds(base, _BPW)], idx_v)
    pltpu.async_copy(table_hbm.at[idx_v], rows_v, sem).wait()
    pltpu.sync_copy(rows_v, out_hbm.at[pl.ds(base, _BPW)])


_sc_gather = pl.kernel(
    _sc_gather_body,
    out_type=jax.ShapeDtypeStruct((B, 2 * H), jnp.float32),
    mesh=plsc.VectorSubcoreMesh(core_axis_name="c", subcore_axis_name="s"),
    scratch_types=[
        pltpu.VMEM((_BPW,), jnp.int32),
        pltpu.VMEM((_BPW, 2 * H), jnp.float32),
        pltpu.SemaphoreType.DMA,
    ],
)


_BLK = 6144


def _proj_body(emb_ref, bias_ref, wt_ref, b_ref, out_ref):
    e = emb_ref[:, :H] + bias_ref[...]
    e1 = jnp.concatenate([e, jnp.ones((B, 1), jnp.float32)], axis=1)
    w1 = jnp.concatenate([wt_ref[...], b_ref[...]], axis=0)
    out_ref[...] = lax.dot_general(
        w1, e1, (((0,), (1,)), ((), ())),
        preferred_element_type=jnp.float32,
    )


@jax.jit
def kernel(x, weight, bias, W_out, b_out):
    x = x.astype(jnp.int32)
    weight_t = jnp.swapaxes(weight, 0, 1)
    w_t = jnp.swapaxes(W_out, 0, 1)

    table_pad = pl.pallas_call(
        _retile_body,
        grid=(pl.cdiv(D, _RBLK),),
        in_specs=[pl.BlockSpec((H, _RBLK), lambda j: (0, j))],
        out_specs=pl.BlockSpec((_RBLK, 2 * H), lambda j: (j, 0)),
        out_shape=jax.ShapeDtypeStruct((D, 2 * H), jnp.float32),
    )(weight_t)

    emb128 = _sc_gather(table_pad, x)

    out_t = pl.pallas_call(
        _proj_body,
        grid=(pl.cdiv(D, _BLK),),
        in_specs=[
            pl.BlockSpec((B, 2 * H), lambda j: (0, 0)),
            pl.BlockSpec((1, H), lambda j: (0, 0)),
            pl.BlockSpec((H, _BLK), lambda j: (0, j)),
            pl.BlockSpec((1, _BLK), lambda j: (0, j)),
        ],
        out_specs=pl.BlockSpec((_BLK, B), lambda j: (j, 0)),
        out_shape=jax.ShapeDtypeStruct((D, B), jnp.float32),
        compiler_params=pltpu.CompilerParams(
            vmem_limit_bytes=120 * 1024 * 1024,
        ),
    )(emb128, bias.reshape(1, H), w_t, b_out.reshape(1, D))
    return jnp.swapaxes(out_t, 0, 1)


# speedup vs baseline: 3.4954x; 1.0069x over previous
import jax
import jax.numpy as jnp
from jax import lax
from jax.experimental import pallas as pl
from jax.experimental.pallas import tpu as pltpu
from jax.experimental.pallas import tpu_sc as plsc

B = 1024
D = 100000
H = 64


_RBLK = 16384


def _retile_body(wt_ref, out_ref):
    t = jnp.transpose(wt_ref[...], (1, 0))
    out_ref[:, :H] = t
    out_ref[:, H:] = jnp.zeros((_RBLK, H), jnp.float32)


_NC = 2
_NS = 16
_NW = _NC * _NS
_BPW = B // _NW


def _sc_gather_body(table_hbm, idx_hbm, out_hbm, idx_v, rows_v, sem):
    wid = lax.axis_index("s") * _NC + lax.axis_index("c")
    base = wid * _BPW
    pltpu.sync_copy(idx_hbm.at[pl.ds(base, _BPW)], idx_v)
    pltpu.async_copy(table_hbm.at[idx_v], rows_v, sem).wait()
    pltpu.sync_copy(rows_v, out_hbm.at[pl.ds(base, _BPW)])


_sc_gather = pl.kernel(
    _sc_gather_body,
    out_type=jax.ShapeDtypeStruct((B, 2 * H), jnp.float32),
    mesh=plsc.VectorSubcoreMesh(core_axis_name="c", subcore_axis_name="s"),
    scratch_types=[
        pltpu.VMEM((_BPW,), jnp.int32),
        pltpu.VMEM((_BPW, 2 * H), jnp.float32),
        pltpu.SemaphoreType.DMA,
    ],
)


_BLK = 4096


def _proj_body(emb_ref, bias_ref, wt_ref, b_ref, out_ref):
    e = emb_ref[:, :H] + bias_ref[...]
    e1 = jnp.concatenate([e, jnp.ones((B, 1), jnp.float32)], axis=1)
    w1 = jnp.concatenate([wt_ref[...], b_ref[...]], axis=0)
    out_ref[...] = lax.dot_general(
        w1, e1, (((0,), (1,)), ((), ())),
        preferred_element_type=jnp.float32,
    )


@jax.jit
def kernel(x, weight, bias, W_out, b_out):
    x = x.astype(jnp.int32)
    weight_t = jnp.swapaxes(weight, 0, 1)
    w_t = jnp.swapaxes(W_out, 0, 1)

    table_pad = pl.pallas_call(
        _retile_body,
        grid=(pl.cdiv(D, _RBLK),),
        in_specs=[pl.BlockSpec((H, _RBLK), lambda j: (0, j))],
        out_specs=pl.BlockSpec((_RBLK, 2 * H), lambda j: (j, 0)),
        out_shape=jax.ShapeDtypeStruct((D, 2 * H), jnp.float32),
    )(weight_t)

    emb128 = _sc_gather(table_pad, x)

    out_t = pl.pallas_call(
        _proj_body,
        grid=(pl.cdiv(D, _BLK),),
        in_specs=[
            pl.BlockSpec((B, 2 * H), lambda j: (0, 0)),
            pl.BlockSpec((1, H), lambda j: (0, 0)),
            pl.BlockSpec((H, _BLK), lambda j: (0, j)),
            pl.BlockSpec((1, _BLK), lambda j: (0, j)),
        ],
        out_specs=pl.BlockSpec((_BLK, B), lambda j: (j, 0)),
        out_shape=jax.ShapeDtypeStruct((D, B), jnp.float32),
        compiler_params=pltpu.CompilerParams(
            vmem_limit_bytes=120 * 1024 * 1024,
        ),
    )(emb128, bias.reshape(1, H), w_t, b_out.reshape(1, D))
    return jnp.swapaxes(out_t, 0, 1)
